# Initial kernel scaffold; baseline (speedup 1.0000x reference)
import jax
import jax.numpy as jnp
from jax.experimental import pallas as pl


def kernel(gt_boxes_select_weight, gt_boxes_batch_ids, batch_num_gt_boxes):
    raise NotImplementedError("write your pallas kernel here")


# speedup vs baseline: 7.0542x; 7.0542x over previous
import functools

import jax
import jax.numpy as jnp
from jax import lax
from jax.experimental import pallas as pl
from jax.experimental.pallas import tpu as pltpu
from jax.experimental.pallas import tpu_sc as plsc

MAX_GT = 4096
HALF = MAX_GT // 2
CHUNK = HALF + 16


def _sc_body(w_hbm, counts_hbm, out_hbm, counts_v, buf_v, out_v):
    wid = lax.axis_index("s") * 2 + lax.axis_index("c")
    row = wid // 2
    half = wid % 2

    pltpu.sync_copy(counts_hbm, counts_v)
    c = counts_v[...]
    start_i = jnp.int32(0)
    count_i = jnp.int32(0)
    for k in range(16):
        ck = c[k]
        start_i = start_i + jnp.where(k < row, ck, 0)
        count_i = count_i + jnp.where(k == row, ck, 0)
    lane = lax.iota(jnp.int32, 16)

    src = start_i + half * HALF
    al = pl.multiple_of((src >> 3) << 3, 8)
    delta = src - al
    pltpu.sync_copy(w_hbm.at[pl.ds(al, CHUNK)], buf_v)

    pbase = half * HALF

    def body(j, carry):
        base = j * 16
        val = plsc.load_gather(buf_v, [base + delta + lane])
        pos = pbase + base + lane
        out_v[pl.ds(base, 16)] = jnp.where(pos < count_i, val, jnp.float32(-1.0))
        return carry

    lax.fori_loop(0, HALF // 16, body, 0)
    pltpu.sync_copy(out_v, out_hbm.at[pl.ds(wid * HALF, HALF)])


def kernel(gt_boxes_select_weight, gt_boxes_batch_ids, batch_num_gt_boxes):
    del gt_boxes_batch_ids
    w = gt_boxes_select_weight
    n_total = w.shape[0]
    b = batch_num_gt_boxes.shape[0]

    n_pad = ((n_total + HALF + CHUNK + 15) // 16) * 16
    w_pad = jnp.concatenate([w, jnp.zeros((n_pad - n_total,), w.dtype)])
    counts = batch_num_gt_boxes.reshape(b).astype(jnp.int32)

    mesh = plsc.VectorSubcoreMesh(core_axis_name="c", subcore_axis_name="s")
    out = pl.kernel(
        _sc_body,
        mesh=mesh,
        out_type=jax.ShapeDtypeStruct((b * MAX_GT,), jnp.float32),
        compiler_params=pltpu.CompilerParams(needs_layout_passes=False),
        scratch_types=[
            pltpu.VMEM((16,), jnp.int32),
            pltpu.VMEM((CHUNK,), jnp.float32),
            pltpu.VMEM((HALF,), jnp.float32),
        ],
    )(w_pad, counts)
    return out.reshape(b, MAX_GT)


# speedup vs baseline: 7.0774x; 1.0033x over previous
import functools

import jax
import jax.numpy as jnp
from jax import lax
from jax.experimental import pallas as pl
from jax.experimental.pallas import tpu as pltpu
from jax.experimental.pallas import tpu_sc as plsc

MAX_GT = 4096
HALF = MAX_GT // 2
CHUNK = HALF + 16
BUF = CHUNK + 32


def _sc_body(n_lim, w_hbm, counts_hbm, out_hbm, counts_v, buf_v, out_v):
    wid = lax.axis_index("s") * 2 + lax.axis_index("c")
    row = wid // 2
    half = wid % 2

    pltpu.sync_copy(counts_hbm, counts_v)
    c = counts_v[...]
    start_i = jnp.int32(0)
    count_i = jnp.int32(0)
    for k in range(16):
        ck = c[k]
        start_i = start_i + jnp.where(k < row, ck, 0)
        count_i = count_i + jnp.where(k == row, ck, 0)

    pbase = half * HALF
    src = start_i + pbase
    w_lo = jnp.minimum((src >> 3) << 3, n_lim)
    w_lo = pl.multiple_of(w_lo, 8)
    delta = src - w_lo
    pltpu.sync_copy(w_hbm.at[pl.ds(w_lo, CHUNK)], buf_v.at[pl.ds(0, CHUNK)])

    rem = jnp.clip(count_i - pbase, 0, HALF)
    ngroups = (rem + 15) >> 4

    lane = lax.iota(jnp.int32, 16)
    vd = delta + lane
    vp = pbase + lane
    neg1 = jnp.full((16,), -1.0, jnp.float32)

    def copy_body(j, carry):
        s = j * 16
        val = plsc.load_gather(buf_v, [s + vd])
        out_v[pl.ds(s, 16)] = jnp.where(s + vp < count_i, val, neg1)
        return carry

    def fill_body(j, carry):
        out_v[pl.ds(j * 16, 16)] = neg1
        return carry

    lax.fori_loop(0, ngroups, copy_body, 0)
    lax.fori_loop(ngroups, HALF // 16, fill_body, 0)

    pltpu.sync_copy(out_v, out_hbm.at[pl.ds(wid * HALF, HALF)])


def kernel(gt_boxes_select_weight, gt_boxes_batch_ids, batch_num_gt_boxes):
    del gt_boxes_batch_ids
    w = gt_boxes_select_weight
    n_total = w.shape[0]
    b = batch_num_gt_boxes.shape[0]
    counts = batch_num_gt_boxes.reshape(b)

    n_lim = ((n_total - CHUNK) // 8) * 8

    mesh = plsc.VectorSubcoreMesh(core_axis_name="c", subcore_axis_name="s")
    out = pl.kernel(
        functools.partial(_sc_body, n_lim),
        mesh=mesh,
        out_type=jax.ShapeDtypeStruct((b * MAX_GT,), jnp.float32),
        compiler_params=pltpu.CompilerParams(needs_layout_passes=False),
        scratch_types=[
            pltpu.VMEM((16,), jnp.int32),
            pltpu.VMEM((BUF,), jnp.float32),
            pltpu.VMEM((HALF,), jnp.float32),
        ],
    )(w, counts)
    return out.reshape(b, MAX_GT)


# speedup vs baseline: 7.1304x; 1.0075x over previous
import functools

import jax
import jax.numpy as jnp
from jax import lax
from jax.experimental import pallas as pl
from jax.experimental.pallas import tpu as pltpu
from jax.experimental.pallas import tpu_sc as plsc

MAX_GT = 4096
HALF = MAX_GT // 2
CHUNK = HALF + 16
BUF = CHUNK + 32


def _sc_body(n_lim, w_hbm, counts_hbm, out_hbm, counts_v, buf_v, out_v, sem):
    wid = lax.axis_index("s") * 2 + lax.axis_index("c")
    row = wid // 2
    half = wid % 2

    cdma = pltpu.async_copy(counts_hbm, counts_v, sem)

    neg1 = jnp.full((16,), -1.0, jnp.float32)

    def fill_body(j, carry):
        out_v[pl.ds(j * 16, 16)] = neg1
        return carry

    lax.fori_loop(0, HALF // 16, fill_body, 0)

    cdma.wait()
    c = counts_v[...]
    start_i = jnp.int32(0)
    count_i = jnp.int32(0)
    for k in range(16):
        ck = c[k]
        start_i = start_i + jnp.where(k < row, ck, 0)
        count_i = count_i + jnp.where(k == row, ck, 0)

    pbase = half * HALF
    src = start_i + pbase
    w_lo = jnp.minimum((src >> 3) << 3, n_lim)
    w_lo = pl.multiple_of(w_lo, 8)
    delta = src - w_lo
    pltpu.sync_copy(w_hbm.at[pl.ds(w_lo, CHUNK)], buf_v.at[pl.ds(0, CHUNK)])

    rem = jnp.clip(count_i - pbase, 0, HALF)
    ngroups = (rem + 15) >> 4

    lane = lax.iota(jnp.int32, 16)
    vd = delta + lane
    vp = pbase + lane

    def copy_body(j, carry):
        s = j * 16
        val = plsc.load_gather(buf_v, [s + vd])
        out_v[pl.ds(s, 16)] = jnp.where(s + vp < count_i, val, neg1)
        return carry

    lax.fori_loop(0, ngroups, copy_body, 0)

    pltpu.sync_copy(out_v, out_hbm.at[pl.ds(wid * HALF, HALF)])


def kernel(gt_boxes_select_weight, gt_boxes_batch_ids, batch_num_gt_boxes):
    del gt_boxes_batch_ids
    w = gt_boxes_select_weight
    n_total = w.shape[0]
    b = batch_num_gt_boxes.shape[0]
    counts = batch_num_gt_boxes.reshape(b)

    n_lim = ((n_total - CHUNK) // 8) * 8

    mesh = plsc.VectorSubcoreMesh(core_axis_name="c", subcore_axis_name="s")
    out = pl.kernel(
        functools.partial(_sc_body, n_lim),
        mesh=mesh,
        out_type=jax.ShapeDtypeStruct((b * MAX_GT,), jnp.float32),
        compiler_params=pltpu.CompilerParams(
            needs_layout_passes=False, skip_device_barrier=True
        ),
        scratch_types=[
            pltpu.VMEM((16,), jnp.int32),
            pltpu.VMEM((BUF,), jnp.float32),
            pltpu.VMEM((HALF,), jnp.float32),
            pltpu.SemaphoreType.DMA,
        ],
    )(w, counts)
    return out.reshape(b, MAX_GT)


# speedup vs baseline: 7.1897x; 1.0083x over previous
import functools

import jax
import jax.numpy as jnp
from jax import lax
from jax.experimental import pallas as pl
from jax.experimental.pallas import tpu as pltpu
from jax.experimental.pallas import tpu_sc as plsc

MAX_GT = 4096
HALF = MAX_GT // 2
CHUNK = HALF + 16
BUF = CHUNK + 32


def _sc_body(n_lim, w_hbm, counts_hbm, out_hbm, counts_v, buf_v, out_v, sem):
    wid = lax.axis_index("s") * 2 + lax.axis_index("c")
    row = wid // 2
    half = wid % 2

    cdma = pltpu.async_copy(counts_hbm, counts_v, sem)

    neg1 = jnp.full((16,), -1.0, jnp.float32)

    @plsc.parallel_loop(0, HALF, step=16, unroll=8)
    def _fill(i):
        out_v[pl.ds(i, 16)] = neg1

    cdma.wait()
    c = counts_v[...]
    start_i = jnp.int32(0)
    count_i = jnp.int32(0)
    for k in range(16):
        ck = c[k]
        start_i = start_i + jnp.where(k < row, ck, 0)
        count_i = count_i + jnp.where(k == row, ck, 0)

    pbase = half * HALF
    src = start_i + pbase
    w_lo = jnp.minimum((src >> 3) << 3, n_lim)
    w_lo = pl.multiple_of(w_lo, 8)
    delta = src - w_lo
    pltpu.sync_copy(w_hbm.at[pl.ds(w_lo, CHUNK)], buf_v.at[pl.ds(0, CHUNK)])

    rem = jnp.clip(count_i - pbase, 0, HALF)
    ngroups = (rem + 15) >> 4

    lane = lax.iota(jnp.int32, 16)
    vd = delta + lane
    vp = pbase + lane

    @plsc.parallel_loop(0, ngroups * 16, step=16, unroll=4)
    def _copy(s):
        val = plsc.load_gather(buf_v, [s + vd])
        out_v[pl.ds(s, 16)] = jnp.where(s + vp < count_i, val, neg1)

    pltpu.sync_copy(out_v, out_hbm.at[pl.ds(wid * HALF, HALF)])


def kernel(gt_boxes_select_weight, gt_boxes_batch_ids, batch_num_gt_boxes):
    del gt_boxes_batch_ids
    w = gt_boxes_select_weight
    n_total = w.shape[0]
    b = batch_num_gt_boxes.shape[0]
    counts = batch_num_gt_boxes.reshape(b)

    n_lim = ((n_total - CHUNK) // 8) * 8

    mesh = plsc.VectorSubcoreMesh(core_axis_name="c", subcore_axis_name="s")
    out = pl.kernel(
        functools.partial(_sc_body, n_lim),
        mesh=mesh,
        out_type=jax.ShapeDtypeStruct((b * MAX_GT,), jnp.float32),
        compiler_params=pltpu.CompilerParams(
            needs_layout_passes=False, skip_device_barrier=True
        ),
        scratch_types=[
            pltpu.VMEM((16,), jnp.int32),
            pltpu.VMEM((BUF,), jnp.float32),
            pltpu.VMEM((HALF,), jnp.float32),
            pltpu.SemaphoreType.DMA,
        ],
    )(w, counts)
    return out.reshape(b, MAX_GT)
